# baseline (speedup 1.0000x reference)
import functools

import jax
import jax.numpy as jnp
from jax import lax
from jax.experimental import pallas as pl
from jax.experimental.pallas import tpu as pltpu
from jax.experimental.pallas import tpu_sc as plsc

B = 1024
D = 128
V = 100000

_NC, _NS = 2, 16
_NW = _NC * _NS
_BPW = B // _NW


@functools.cache
def _make_sc_gather():
    mesh = plsc.VectorSubcoreMesh(
        core_axis_name="c", subcore_axis_name="s",
        num_cores=_NC, num_subcores=_NS)

    half = _BPW // 2

    @functools.partial(
        pl.kernel,
        out_type=jax.ShapeDtypeStruct((B, D), jnp.float32),
        mesh=mesh,
        scratch_types=[
            pltpu.VMEM((_BPW,), jnp.int32),
            pltpu.VMEM((_BPW, D), jnp.float32),
            pltpu.SemaphoreType.DMA,
            pltpu.SemaphoreType.DMA,
            pltpu.SemaphoreType.DMA,
        ],
    )
    def _sc_gather(idx_hbm, table_hbm, out_hbm, idx_v, rows_v, g0, g1, wb):
        wid = lax.axis_index("s") * _NC + lax.axis_index("c")
        base = wid * _BPW
        pltpu.sync_copy(idx_hbm.at[pl.ds(base, _BPW)], idx_v)
        c0 = pltpu.async_copy(
            table_hbm.at[idx_v.at[pl.ds(0, half)]],
            rows_v.at[pl.ds(0, half)], g0)
        c1 = pltpu.async_copy(
            table_hbm.at[idx_v.at[pl.ds(half, half)]],
            rows_v.at[pl.ds(half, half)], g1)
        c0.wait()
        w0 = pltpu.async_copy(
            rows_v.at[pl.ds(0, half)], out_hbm.at[pl.ds(base, half)], wb)
        c1.wait()
        w1 = pltpu.async_copy(
            rows_v.at[pl.ds(half, half)],
            out_hbm.at[pl.ds(base + half, half)], wb)
        w0.wait()
        w1.wait()

    return _sc_gather


_VT = 6144
_NVT = (V + _VT - 1) // _VT


def _mm_body(w_ref, e_ref, b_ref, out_ref):
    bias_col = jnp.transpose(b_ref[...])
    out_ref[...] = (
        jax.lax.dot_general(
            w_ref[...], e_ref[...], (((1,), (1,)), ((), ())),
            preferred_element_type=jnp.float32,
        )
        + bias_col
    )


_mm = pl.pallas_call(
    _mm_body,
    grid=(_NVT,),
    in_specs=[
        pl.BlockSpec((_VT, D), lambda v: (v, 0)),
        pl.BlockSpec((B, D), lambda v: (0, 0)),
        pl.BlockSpec((1, _VT), lambda v: (0, v)),
    ],
    out_specs=pl.BlockSpec((_VT, B), lambda v: (v, 0)),
    out_shape=jax.ShapeDtypeStruct((V, B), jnp.float32),
    compiler_params=pltpu.CompilerParams(vmem_limit_bytes=63 * 2**20),
)


@jax.jit
def kernel(target_idx, emb_table, W, b):
    embed = _make_sc_gather()(target_idx, emb_table)
    out_t = _mm(W.T, embed, b.reshape(1, V))
    return out_t.T
